# Initial kernel scaffold; baseline (speedup 1.0000x reference)
import jax
import jax.numpy as jnp
from jax import lax
from jax.experimental import pallas as pl
from jax.experimental.pallas import tpu as pltpu
from jax.experimental.pallas import tpu_sc as plsc

N_NODES = 10000
N_PAD = 10240
E = 160000
E_ROWS = 1280
E_PAD = E_ROWS * 128
IN_CH = 256
HALF = 128
HIDDEN = 512
OUT_CH = 2
NC = 2
NS = 16
NW = NC * NS
ROWS_A = E_ROWS // NW
ROWS_C = E_ROWS // NS
NPT = N_PAD // NS
RB = 2048
GRID = 5

_mesh = plsc.VectorSubcoreMesh(core_axis_name="c", subcore_axis_name="s")


def _deg_body(dst_hbm, out_hbm, idx_v, hist_v):
    cid = lax.axis_index("c")
    sid = lax.axis_index("s")
    wid = sid * NC + cid
    zeros16 = jnp.zeros((16,), jnp.float32)

    def zb(i, _):
        hist_v[pl.ds(i * 16, 16)] = zeros16
        return 0

    lax.fori_loop(0, N_PAD // 16, zb, 0)
    pltpu.sync_copy(dst_hbm.at[pl.ds(wid * ROWS_A, ROWS_A)], idx_v)
    ones16 = jnp.ones((16,), jnp.float32)

    def body(j, _):
        for k in range(8):
            idx = idx_v[j, pl.ds(k * 16, 16)]
            plsc.addupdate_scatter(hist_v, [idx], ones16)
        return 0

    lax.fori_loop(0, ROWS_A, body, 0)
    pltpu.sync_copy(hist_v, out_hbm.at[wid])


_deg_call = pl.kernel(
    _deg_body,
    out_type=jax.ShapeDtypeStruct((NW, N_PAD), jnp.float32),
    mesh=_mesh,
    scratch_types=[
        pltpu.VMEM((ROWS_A, 128), jnp.int32),
        pltpu.VMEM((N_PAD,), jnp.float32),
    ],
)


def _prep_body(x_ref, degp_ref, dis_ref, y0_ref, y1_ref):
    deg = jnp.sum(degp_ref[...], axis=0) + 1.0
    dis = lax.rsqrt(deg)[:, None]
    dis_ref[...] = dis
    xb = x_ref[...]
    y0_ref[...] = xb[:, :HALF] * dis
    y1_ref[...] = xb[:, HALF:] * dis


_prep_call = pl.pallas_call(
    _prep_body,
    grid=(GRID,),
    in_specs=[
        pl.BlockSpec((RB, IN_CH), lambda i: (i, 0)),
        pl.BlockSpec((NW, RB), lambda i: (0, i)),
    ],
    out_specs=[
        pl.BlockSpec((RB, 1), lambda i: (i, 0)),
        pl.BlockSpec((RB, HALF), lambda i: (i, 0)),
        pl.BlockSpec((RB, HALF), lambda i: (i, 0)),
    ],
    out_shape=[
        jax.ShapeDtypeStruct((N_PAD, 1), jnp.float32),
        jax.ShapeDtypeStruct((N_PAD, HALF), jnp.float32),
        jax.ShapeDtypeStruct((N_PAD, HALF), jnp.float32),
    ],
)


def _agg_body(src_hbm, dst_hbm, y0_hbm, y1_hbm, agg0_hbm, agg1_hbm,
              srcv, dstv, rows_v, acc_sh, sem):
    cid = lax.axis_index("c")
    sid = lax.axis_index("s")
    base = sid * ROWS_C
    pltpu.sync_copy(src_hbm.at[pl.ds(base, ROWS_C)], srcv)
    pltpu.sync_copy(dst_hbm.at[pl.ds(base, ROWS_C)], dstv)

    zeros16 = jnp.zeros((16,), jnp.float32)

    def zb(i, _):
        for k in range(8):
            rows_v[i, pl.ds(k * 16, 16)] = zeros16
        return 0

    lax.fori_loop(0, 128, zb, 0)
    for k in range(NPT // 128):
        pltpu.sync_copy(rows_v, acc_sh.at[pl.ds(sid * NPT + k * 128, 128)])
    plsc.subcore_barrier()

    def do_core(y_hbm):
        def body(j, _):
            pltpu.async_copy(y_hbm.at[srcv.at[j]], rows_v, sem).wait()
            pltpu.sync_copy(rows_v, acc_sh.at[dstv.at[j]], add=True)
            return 0

        lax.fori_loop(0, ROWS_C, body, 0)

    @pl.when(cid == 0)
    def _():
        do_core(y0_hbm)

    @pl.when(cid == 1)
    def _():
        do_core(y1_hbm)

    plsc.subcore_barrier()
    nb = sid * NPT

    @pl.when(cid == 0)
    def _():
        pltpu.sync_copy(acc_sh.at[pl.ds(nb, NPT)], agg0_hbm.at[pl.ds(nb, NPT)])

    @pl.when(cid == 1)
    def _():
        pltpu.sync_copy(acc_sh.at[pl.ds(nb, NPT)], agg1_hbm.at[pl.ds(nb, NPT)])


_agg_call = pl.kernel(
    _agg_body,
    out_type=(
        jax.ShapeDtypeStruct((N_PAD, HALF), jnp.float32),
        jax.ShapeDtypeStruct((N_PAD, HALF), jnp.float32),
    ),
    mesh=_mesh,
    scratch_types=[
        pltpu.VMEM((ROWS_C, 128), jnp.int32),
        pltpu.VMEM((ROWS_C, 128), jnp.int32),
        pltpu.VMEM((128, HALF), jnp.float32),
        pltpu.VMEM_SHARED((N_PAD, HALF), jnp.float32),
        pltpu.SemaphoreType.DMA,
    ],
)


def _main_body(dis_ref, agg0_ref, agg1_ref, y0_ref, y1_ref,
               w1_ref, b1_ref, w2_ref, u_ref):
    dis = dis_ref[...]
    z0 = (agg0_ref[...] + y0_ref[...]) * dis
    z1 = (agg1_ref[...] + y1_ref[...]) * dis
    w1 = w1_ref[...]
    h = jnp.dot(z0, w1[:HALF], preferred_element_type=jnp.float32)
    h = h + jnp.dot(z1, w1[HALF:], preferred_element_type=jnp.float32)
    h = jnp.maximum(h + b1_ref[...], 0.0)
    u_ref[...] = jnp.dot(h, w2_ref[...], preferred_element_type=jnp.float32) * dis


_main_call = pl.pallas_call(
    _main_body,
    grid=(GRID,),
    in_specs=[
        pl.BlockSpec((RB, 1), lambda i: (i, 0)),
        pl.BlockSpec((RB, HALF), lambda i: (i, 0)),
        pl.BlockSpec((RB, HALF), lambda i: (i, 0)),
        pl.BlockSpec((RB, HALF), lambda i: (i, 0)),
        pl.BlockSpec((RB, HALF), lambda i: (i, 0)),
        pl.BlockSpec((IN_CH, HIDDEN), lambda i: (0, 0)),
        pl.BlockSpec((1, HIDDEN), lambda i: (0, 0)),
        pl.BlockSpec((HIDDEN, OUT_CH), lambda i: (0, 0)),
    ],
    out_specs=pl.BlockSpec((RB, OUT_CH), lambda i: (i, 0)),
    out_shape=jax.ShapeDtypeStruct((N_PAD, OUT_CH), jnp.float32),
)


def _fin_body(src_hbm, dst_hbm, u_hbm, dis_hbm, b2_hbm, out_hbm,
              srcv, dstv, u_v, p_v, tmp_v, o_v, b2_v, dis_v, sp_sh):
    cid = lax.axis_index("c")
    sid = lax.axis_index("s")

    @pl.when(cid == 0)
    def _():
        pltpu.sync_copy(src_hbm.at[pl.ds(sid * ROWS_C, ROWS_C)], srcv)
        pltpu.sync_copy(dst_hbm.at[pl.ds(sid * ROWS_C, ROWS_C)], dstv)
        pltpu.sync_copy(u_hbm, u_v)
        pltpu.sync_copy(dis_hbm, dis_v)
        pltpu.sync_copy(b2_hbm, b2_v)
        zeros16 = jnp.zeros((16,), jnp.float32)

        def zb(i, _):
            p_v[pl.ds(i * 16, 16)] = zeros16
            return 0

        lax.fori_loop(0, (2 * N_PAD) // 16, zb, 0)
        ones = jnp.ones((16,), jnp.int32)

        def body(j, _):
            for k in range(8):
                s16 = srcv[j, pl.ds(k * 16, 16)]
                d16 = dstv[j, pl.ds(k * 16, 16)]
                s2 = s16 + s16
                d2 = d16 + d16
                g0 = plsc.load_gather(u_v, [s2])
                g1 = plsc.load_gather(u_v, [s2 + ones])
                plsc.addupdate_scatter(p_v, [d2], g0)
                plsc.addupdate_scatter(p_v, [d2 + ones], g1)
            return 0

        lax.fori_loop(0, ROWS_C, body, 0)
        pltpu.sync_copy(p_v, sp_sh.at[sid])
        plsc.subcore_barrier()

        fbase = sid * 2 * NPT

        def zb2(i, _):
            o_v[pl.ds(i * 16, 16)] = zeros16
            return 0

        lax.fori_loop(0, (2 * NPT) // 16, zb2, 0)
        for t in range(NS):
            pltpu.sync_copy(sp_sh.at[t, pl.ds(fbase, 2 * NPT)], tmp_v)

            def ab(i, _):
                o_v[pl.ds(i * 16, 16)] += tmp_v[pl.ds(i * 16, 16)]
                return 0

            lax.fori_loop(0, (2 * NPT) // 16, ab, 0)

        b2v = b2_v[...]
        iota = lax.iota(jnp.int32, 16)
        half_iota = lax.shift_right_logical(iota, 1)
        nbase = sid * NPT

        def fb(i, _):
            pv = o_v[pl.ds(i * 16, 16)]
            uv = u_v[pl.ds(fbase + i * 16, 16)]
            dpair = plsc.load_gather(dis_v, [nbase + i * 8 + half_iota])
            o_v[pl.ds(i * 16, 16)] = dpair * (pv + uv) + b2v
            return 0

        lax.fori_loop(0, (2 * NPT) // 16, fb, 0)
        pltpu.sync_copy(o_v, out_hbm.at[pl.ds(fbase, 2 * NPT)])


_fin_call = pl.kernel(
    _fin_body,
    out_type=jax.ShapeDtypeStruct((2 * N_PAD,), jnp.float32),
    mesh=_mesh,
    scratch_types=[
        pltpu.VMEM((ROWS_C, 128), jnp.int32),
        pltpu.VMEM((ROWS_C, 128), jnp.int32),
        pltpu.VMEM((2 * N_PAD,), jnp.float32),
        pltpu.VMEM((2 * N_PAD,), jnp.float32),
        pltpu.VMEM((2 * NPT,), jnp.float32),
        pltpu.VMEM((2 * NPT,), jnp.float32),
        pltpu.VMEM((16,), jnp.float32),
        pltpu.VMEM((N_PAD,), jnp.float32),
        pltpu.VMEM_SHARED((NS, 2 * N_PAD), jnp.float32),
    ],
)


def kernel(x, edge_index, W1, b1, W2, b2):
    ei = edge_index.astype(jnp.int32)
    src, dst = ei[0], ei[1]
    padi = jnp.arange(E_PAD - E, dtype=jnp.int32)
    pad_src = padi % N_NODES
    pad_dst = N_NODES + padi % (N_PAD - N_NODES)
    srcp = jnp.concatenate([src, pad_src]).reshape(E_ROWS, 128)
    dstp = jnp.concatenate([dst, pad_dst]).reshape(E_ROWS, 128)

    degp = _deg_call(dstp)
    dis, y0, y1 = _prep_call(x, degp)
    agg0, agg1 = _agg_call(srcp, dstp, y0, y1)
    u = _main_call(dis, agg0, agg1, y0, y1, W1, b1.reshape(1, HIDDEN), W2)
    out_flat = _fin_call(srcp, dstp, u.reshape(2 * N_PAD),
                         dis.reshape(N_PAD), jnp.tile(b2, 8))
    return out_flat.reshape(N_PAD, OUT_CH)[:N_NODES]


# speedup vs baseline: 24.0544x; 24.0544x over previous
import jax
import jax.numpy as jnp
from jax import lax
from jax.experimental import pallas as pl
from jax.experimental.pallas import tpu as pltpu
from jax.experimental.pallas import tpu_sc as plsc

N_NODES = 10000
N_PAD = 10240
E = 160000
E_ROWS = 1280
E_PAD = E_ROWS * 128
IN_CH = 256
HALF = 128
HIDDEN = 512
OUT_CH = 2
NC = 2
NS = 16
NW = NC * NS
ROWS_A = E_ROWS // NW
ROWS_C = E_ROWS // NS
NPT = N_PAD // NS
RB = 2048
GRID = 5

_mesh = plsc.VectorSubcoreMesh(core_axis_name="c", subcore_axis_name="s")
_sc_params = pltpu.CompilerParams(needs_layout_passes=False)


def _deg_body(dst_hbm, out_hbm, idx_v, hist_v):
    cid = lax.axis_index("c")
    sid = lax.axis_index("s")
    wid = sid * NC + cid
    zeros16 = jnp.zeros((16,), jnp.float32)

    def zb(i, _):
        hist_v[pl.ds(i * 16, 16)] = zeros16
        return 0

    lax.fori_loop(0, N_PAD // 16, zb, 0)
    pltpu.sync_copy(dst_hbm.at[pl.ds(wid * ROWS_A, ROWS_A)], idx_v)
    ones16 = jnp.ones((16,), jnp.float32)

    def body(j, _):
        for k in range(8):
            idx = idx_v[j, pl.ds(k * 16, 16)]
            plsc.addupdate_scatter(hist_v, [idx], ones16)
        return 0

    lax.fori_loop(0, ROWS_A, body, 0)
    pltpu.sync_copy(hist_v, out_hbm.at[wid])


_deg_call = pl.kernel(
    _deg_body,
    out_type=jax.ShapeDtypeStruct((NW, N_PAD), jnp.float32),
    mesh=_mesh,
    scratch_types=[
        pltpu.VMEM((ROWS_A, 128), jnp.int32),
        pltpu.VMEM((N_PAD,), jnp.float32),
    ],
    compiler_params=_sc_params,
)


def _prep_body(x_ref, degp_ref, dis_ref, y0_ref, y1_ref):
    deg = jnp.sum(degp_ref[...], axis=0) + 1.0
    dis = lax.rsqrt(deg)[:, None]
    dis_ref[...] = dis
    xb = x_ref[...]
    y0_ref[...] = xb[:, :HALF] * dis
    y1_ref[...] = xb[:, HALF:] * dis


_prep_call = pl.pallas_call(
    _prep_body,
    grid=(GRID,),
    in_specs=[
        pl.BlockSpec((RB, IN_CH), lambda i: (i, 0)),
        pl.BlockSpec((NW, RB), lambda i: (0, i)),
    ],
    out_specs=[
        pl.BlockSpec((RB, 1), lambda i: (i, 0)),
        pl.BlockSpec((RB, HALF), lambda i: (i, 0)),
        pl.BlockSpec((RB, HALF), lambda i: (i, 0)),
    ],
    out_shape=[
        jax.ShapeDtypeStruct((N_PAD, 1), jnp.float32),
        jax.ShapeDtypeStruct((N_PAD, HALF), jnp.float32),
        jax.ShapeDtypeStruct((N_PAD, HALF), jnp.float32),
    ],
)


def _agg_body(src_hbm, dst_hbm, y0_hbm, y1_hbm, agg0_hbm, agg1_hbm,
              srcv, dstv, rows_v, acc_sh, sem):
    cid = lax.axis_index("c")
    sid = lax.axis_index("s")
    base = sid * ROWS_C
    pltpu.sync_copy(src_hbm.at[pl.ds(base, ROWS_C)], srcv)
    pltpu.sync_copy(dst_hbm.at[pl.ds(base, ROWS_C)], dstv)

    zeros16 = jnp.zeros((16,), jnp.float32)

    def zb(i, _):
        for k in range(8):
            rows_v[i, pl.ds(k * 16, 16)] = zeros16
        return 0

    lax.fori_loop(0, 128, zb, 0)
    for k in range(NPT // 128):
        pltpu.sync_copy(rows_v, acc_sh.at[pl.ds(sid * NPT + k * 128, 128)])
    plsc.subcore_barrier()

    def do_core(y_hbm):
        def body(j, _):
            pltpu.async_copy(y_hbm.at[srcv.at[j]], rows_v, sem).wait()
            pltpu.sync_copy(rows_v, acc_sh.at[dstv.at[j]], add=True)
            return 0

        lax.fori_loop(0, ROWS_C, body, 0)

    @pl.when(cid == 0)
    def _():
        do_core(y0_hbm)

    @pl.when(cid == 1)
    def _():
        do_core(y1_hbm)

    plsc.subcore_barrier()
    nb = sid * NPT

    @pl.when(cid == 0)
    def _():
        pltpu.sync_copy(acc_sh.at[pl.ds(nb, NPT)], agg0_hbm.at[pl.ds(nb, NPT)])

    @pl.when(cid == 1)
    def _():
        pltpu.sync_copy(acc_sh.at[pl.ds(nb, NPT)], agg1_hbm.at[pl.ds(nb, NPT)])


_agg_call = pl.kernel(
    _agg_body,
    out_type=(
        jax.ShapeDtypeStruct((N_PAD, HALF), jnp.float32),
        jax.ShapeDtypeStruct((N_PAD, HALF), jnp.float32),
    ),
    mesh=_mesh,
    scratch_types=[
        pltpu.VMEM((ROWS_C, 128), jnp.int32),
        pltpu.VMEM((ROWS_C, 128), jnp.int32),
        pltpu.VMEM((128, HALF), jnp.float32),
        pltpu.VMEM_SHARED((N_PAD, HALF), jnp.float32),
        pltpu.SemaphoreType.DMA,
    ],
    compiler_params=_sc_params,
)


def _main_body(dis_ref, agg0_ref, agg1_ref, y0_ref, y1_ref,
               w1_ref, b1_ref, w2_ref, u_ref):
    dis = dis_ref[...]
    z0 = (agg0_ref[...] + y0_ref[...]) * dis
    z1 = (agg1_ref[...] + y1_ref[...]) * dis
    w1 = w1_ref[...]
    h = jnp.dot(z0, w1[:HALF], preferred_element_type=jnp.float32)
    h = h + jnp.dot(z1, w1[HALF:], preferred_element_type=jnp.float32)
    h = jnp.maximum(h + b1_ref[...], 0.0)
    u_ref[...] = jnp.dot(h, w2_ref[...], preferred_element_type=jnp.float32) * dis


_main_call = pl.pallas_call(
    _main_body,
    grid=(GRID,),
    in_specs=[
        pl.BlockSpec((RB, 1), lambda i: (i, 0)),
        pl.BlockSpec((RB, HALF), lambda i: (i, 0)),
        pl.BlockSpec((RB, HALF), lambda i: (i, 0)),
        pl.BlockSpec((RB, HALF), lambda i: (i, 0)),
        pl.BlockSpec((RB, HALF), lambda i: (i, 0)),
        pl.BlockSpec((IN_CH, HIDDEN), lambda i: (0, 0)),
        pl.BlockSpec((1, HIDDEN), lambda i: (0, 0)),
        pl.BlockSpec((HIDDEN, OUT_CH), lambda i: (0, 0)),
    ],
    out_specs=pl.BlockSpec((RB, OUT_CH), lambda i: (i, 0)),
    out_shape=jax.ShapeDtypeStruct((N_PAD, OUT_CH), jnp.float32),
)


def _fin_body(src_hbm, dst_hbm, u_hbm, dis_hbm, b2_hbm, out_hbm,
              srcv, dstv, u_v, p_v, tmp_v, o_v, b2_v, dis_v, sp_sh):
    cid = lax.axis_index("c")
    sid = lax.axis_index("s")

    @pl.when(cid == 0)
    def _():
        pltpu.sync_copy(src_hbm.at[pl.ds(sid * ROWS_C, ROWS_C)], srcv)
        pltpu.sync_copy(dst_hbm.at[pl.ds(sid * ROWS_C, ROWS_C)], dstv)
        pltpu.sync_copy(u_hbm, u_v)
        pltpu.sync_copy(dis_hbm, dis_v)
        pltpu.sync_copy(b2_hbm, b2_v)
        zeros16 = jnp.zeros((16,), jnp.float32)

        def zb(i, _):
            p_v[pl.ds(i * 16, 16)] = zeros16
            return 0

        lax.fori_loop(0, (2 * N_PAD) // 16, zb, 0)
        ones = jnp.ones((16,), jnp.int32)

        def body(j, _):
            for k in range(8):
                s16 = srcv[j, pl.ds(k * 16, 16)]
                d16 = dstv[j, pl.ds(k * 16, 16)]
                s2 = s16 + s16
                d2 = d16 + d16
                g0 = plsc.load_gather(u_v, [s2])
                g1 = plsc.load_gather(u_v, [s2 + ones])
                plsc.addupdate_scatter(p_v, [d2], g0)
                plsc.addupdate_scatter(p_v, [d2 + ones], g1)
            return 0

        lax.fori_loop(0, ROWS_C, body, 0)
        pltpu.sync_copy(p_v, sp_sh.at[sid])
        plsc.subcore_barrier()

        fbase = sid * 2 * NPT

        def zb2(i, _):
            o_v[pl.ds(i * 16, 16)] = zeros16
            return 0

        lax.fori_loop(0, (2 * NPT) // 16, zb2, 0)
        for t in range(NS):
            pltpu.sync_copy(sp_sh.at[t, pl.ds(fbase, 2 * NPT)], tmp_v)

            def ab(i, _):
                o_v[pl.ds(i * 16, 16)] += tmp_v[pl.ds(i * 16, 16)]
                return 0

            lax.fori_loop(0, (2 * NPT) // 16, ab, 0)

        b2v = b2_v[...]
        iota = lax.iota(jnp.int32, 16)
        half_iota = lax.shift_right_logical(iota, 1)
        nbase = sid * NPT

        def fb(i, _):
            pv = o_v[pl.ds(i * 16, 16)]
            uv = u_v[pl.ds(fbase + i * 16, 16)]
            dpair = plsc.load_gather(dis_v, [nbase + i * 8 + half_iota])
            o_v[pl.ds(i * 16, 16)] = dpair * (pv + uv) + b2v
            return 0

        lax.fori_loop(0, (2 * NPT) // 16, fb, 0)
        pltpu.sync_copy(o_v, out_hbm.at[pl.ds(fbase, 2 * NPT)])


_fin_call = pl.kernel(
    _fin_body,
    out_type=jax.ShapeDtypeStruct((2 * N_PAD,), jnp.float32),
    mesh=_mesh,
    scratch_types=[
        pltpu.VMEM((ROWS_C, 128), jnp.int32),
        pltpu.VMEM((ROWS_C, 128), jnp.int32),
        pltpu.VMEM((2 * N_PAD,), jnp.float32),
        pltpu.VMEM((2 * N_PAD,), jnp.float32),
        pltpu.VMEM((2 * NPT,), jnp.float32),
        pltpu.VMEM((2 * NPT,), jnp.float32),
        pltpu.VMEM((16,), jnp.float32),
        pltpu.VMEM((N_PAD,), jnp.float32),
        pltpu.VMEM_SHARED((NS, 2 * N_PAD), jnp.float32),
    ],
    compiler_params=_sc_params,
)


def kernel(x, edge_index, W1, b1, W2, b2):
    ei = edge_index.astype(jnp.int32)
    src, dst = ei[0], ei[1]
    padi = jnp.arange(E_PAD - E, dtype=jnp.int32)
    pad_src = padi % N_NODES
    pad_dst = N_NODES + padi % (N_PAD - N_NODES)
    srcp = jnp.concatenate([src, pad_src]).reshape(E_ROWS, 128)
    dstp = jnp.concatenate([dst, pad_dst]).reshape(E_ROWS, 128)

    degp = _deg_call(dstp)
    dis, y0, y1 = _prep_call(x, degp)
    agg0, agg1 = _agg_call(srcp, dstp, y0, y1)
    u = _main_call(dis, agg0, agg1, y0, y1, W1, b1.reshape(1, HIDDEN), W2)
    out_flat = _fin_call(srcp, dstp, u.reshape(2 * N_PAD),
                         dis.reshape(N_PAD), jnp.tile(b2, 8))
    return out_flat.reshape(N_PAD, OUT_CH)[:N_NODES]


# speedup vs baseline: 28.7192x; 1.1939x over previous
import jax
import jax.numpy as jnp
from jax import lax
from jax.experimental import pallas as pl
from jax.experimental.pallas import tpu as pltpu
from jax.experimental.pallas import tpu_sc as plsc

N_NODES = 10000
N_PAD = 10240
E = 160000
E_ROWS = 1280
E_PAD = E_ROWS * 128
IN_CH = 256
HALF = 128
HIDDEN = 512
OUT_CH = 2
NC = 2
NS = 16
NW = NC * NS
ROWS_A = E_ROWS // NW
ROWS_C = E_ROWS // NS
NPT = N_PAD // NS
RB = 2048
GRID = 5

_mesh = plsc.VectorSubcoreMesh(core_axis_name="c", subcore_axis_name="s")
_sc_params = pltpu.CompilerParams(needs_layout_passes=False)


def _deg_body(dst_hbm, out_hbm, idx_v, hist_v):
    cid = lax.axis_index("c")
    sid = lax.axis_index("s")
    wid = sid * NC + cid
    zeros16 = jnp.zeros((16,), jnp.float32)

    def zb(i, _):
        hist_v[pl.ds(i * 16, 16)] = zeros16
        return 0

    lax.fori_loop(0, N_PAD // 16, zb, 0)
    pltpu.sync_copy(dst_hbm.at[pl.ds(wid * ROWS_A, ROWS_A)], idx_v)
    ones16 = jnp.ones((16,), jnp.float32)

    def body(j, _):
        for k in range(8):
            idx = idx_v[j, pl.ds(k * 16, 16)]
            plsc.addupdate_scatter(hist_v, [idx], ones16)
        return 0

    lax.fori_loop(0, ROWS_A, body, 0)
    pltpu.sync_copy(hist_v, out_hbm.at[wid])


_deg_call = pl.kernel(
    _deg_body,
    out_type=jax.ShapeDtypeStruct((NW, N_PAD), jnp.float32),
    mesh=_mesh,
    scratch_types=[
        pltpu.VMEM((ROWS_A, 128), jnp.int32),
        pltpu.VMEM((N_PAD,), jnp.float32),
    ],
    compiler_params=_sc_params,
)


def _prep_body(x_ref, degp_ref, dis_ref, y0_ref, y1_ref):
    deg = jnp.sum(degp_ref[...], axis=0) + 1.0
    dis = lax.rsqrt(deg)[:, None]
    dis_ref[...] = dis
    xb = x_ref[...]
    y0_ref[...] = xb[:, :HALF] * dis
    y1_ref[...] = xb[:, HALF:] * dis


_prep_call = pl.pallas_call(
    _prep_body,
    grid=(GRID,),
    in_specs=[
        pl.BlockSpec((RB, IN_CH), lambda i: (i, 0)),
        pl.BlockSpec((NW, RB), lambda i: (0, i)),
    ],
    out_specs=[
        pl.BlockSpec((RB, 1), lambda i: (i, 0)),
        pl.BlockSpec((RB, HALF), lambda i: (i, 0)),
        pl.BlockSpec((RB, HALF), lambda i: (i, 0)),
    ],
    out_shape=[
        jax.ShapeDtypeStruct((N_PAD, 1), jnp.float32),
        jax.ShapeDtypeStruct((N_PAD, HALF), jnp.float32),
        jax.ShapeDtypeStruct((N_PAD, HALF), jnp.float32),
    ],
)


def _agg_body(src_hbm, dst_hbm, y0_hbm, y1_hbm, agg0_hbm, agg1_hbm,
              srcv, dstv, rows_v, rows_v2, acc_sh, sem0, sem1):
    cid = lax.axis_index("c")
    sid = lax.axis_index("s")
    base = sid * ROWS_C
    hrows = ROWS_C // 2

    zeros16 = jnp.zeros((16,), jnp.float32)

    def zb(i, _):
        for k in range(8):
            rows_v[i, pl.ds(k * 16, 16)] = zeros16
        return 0

    lax.fori_loop(0, 128, zb, 0)
    for k in range(NPT // 128):
        pltpu.sync_copy(rows_v, acc_sh.at[pl.ds(sid * NPT + k * 128, 128)])
    plsc.subcore_barrier()

    def do_core(y_hbm):
        for h in range(2):
            pltpu.sync_copy(src_hbm.at[pl.ds(base + h * hrows, hrows)], srcv)
            pltpu.sync_copy(dst_hbm.at[pl.ds(base + h * hrows, hrows)], dstv)
            pltpu.async_copy(y_hbm.at[srcv.at[0]], rows_v, sem0)

            def body(ji, _):
                j0 = ji * 2
                pltpu.make_async_copy(y_hbm.at[srcv.at[j0]], rows_v, sem0).wait()
                pltpu.async_copy(y_hbm.at[srcv.at[j0 + 1]], rows_v2, sem1)
                pltpu.sync_copy(rows_v, acc_sh.at[dstv.at[j0]], add=True)
                pltpu.make_async_copy(y_hbm.at[srcv.at[j0]], rows_v2, sem1).wait()

                @pl.when(ji < hrows // 2 - 1)
                def _():
                    pltpu.async_copy(y_hbm.at[srcv.at[j0 + 2]], rows_v, sem0)

                pltpu.sync_copy(rows_v2, acc_sh.at[dstv.at[j0 + 1]], add=True)
                return 0

            lax.fori_loop(0, hrows // 2, body, 0)

    @pl.when(cid == 0)
    def _():
        do_core(y0_hbm)

    @pl.when(cid == 1)
    def _():
        do_core(y1_hbm)

    plsc.subcore_barrier()
    nb = sid * NPT

    @pl.when(cid == 0)
    def _():
        pltpu.sync_copy(acc_sh.at[pl.ds(nb, NPT)], agg0_hbm.at[pl.ds(nb, NPT)])

    @pl.when(cid == 1)
    def _():
        pltpu.sync_copy(acc_sh.at[pl.ds(nb, NPT)], agg1_hbm.at[pl.ds(nb, NPT)])


_agg_call = pl.kernel(
    _agg_body,
    out_type=(
        jax.ShapeDtypeStruct((N_PAD, HALF), jnp.float32),
        jax.ShapeDtypeStruct((N_PAD, HALF), jnp.float32),
    ),
    mesh=_mesh,
    scratch_types=[
        pltpu.VMEM((ROWS_C // 2, 128), jnp.int32),
        pltpu.VMEM((ROWS_C // 2, 128), jnp.int32),
        pltpu.VMEM((128, HALF), jnp.float32),
        pltpu.VMEM((128, HALF), jnp.float32),
        pltpu.VMEM_SHARED((N_PAD, HALF), jnp.float32),
        pltpu.SemaphoreType.DMA,
        pltpu.SemaphoreType.DMA,
    ],
    compiler_params=_sc_params,
)


def _main_body(dis_ref, agg0_ref, agg1_ref, y0_ref, y1_ref,
               w1_ref, b1_ref, w2_ref, u_ref):
    dis = dis_ref[...]
    z0 = (agg0_ref[...] + y0_ref[...]) * dis
    z1 = (agg1_ref[...] + y1_ref[...]) * dis
    w1 = w1_ref[...]
    h = jnp.dot(z0, w1[:HALF], preferred_element_type=jnp.float32)
    h = h + jnp.dot(z1, w1[HALF:], preferred_element_type=jnp.float32)
    h = jnp.maximum(h + b1_ref[...], 0.0)
    u_ref[...] = jnp.dot(h, w2_ref[...], preferred_element_type=jnp.float32) * dis


_main_call = pl.pallas_call(
    _main_body,
    grid=(GRID,),
    in_specs=[
        pl.BlockSpec((RB, 1), lambda i: (i, 0)),
        pl.BlockSpec((RB, HALF), lambda i: (i, 0)),
        pl.BlockSpec((RB, HALF), lambda i: (i, 0)),
        pl.BlockSpec((RB, HALF), lambda i: (i, 0)),
        pl.BlockSpec((RB, HALF), lambda i: (i, 0)),
        pl.BlockSpec((IN_CH, HIDDEN), lambda i: (0, 0)),
        pl.BlockSpec((1, HIDDEN), lambda i: (0, 0)),
        pl.BlockSpec((HIDDEN, OUT_CH), lambda i: (0, 0)),
    ],
    out_specs=pl.BlockSpec((RB, OUT_CH), lambda i: (i, 0)),
    out_shape=jax.ShapeDtypeStruct((N_PAD, OUT_CH), jnp.float32),
)


def _fin_body(src_hbm, dst_hbm, u_hbm, dis_hbm, b2_hbm, out_hbm,
              srcv, dstv, u_v, p_v, tmp_v, o_v, b2_v, dis_v, sp_sh):
    cid = lax.axis_index("c")
    sid = lax.axis_index("s")

    @pl.when(cid == 0)
    def _():
        pltpu.sync_copy(src_hbm.at[pl.ds(sid * ROWS_C, ROWS_C)], srcv)
        pltpu.sync_copy(dst_hbm.at[pl.ds(sid * ROWS_C, ROWS_C)], dstv)
        pltpu.sync_copy(u_hbm, u_v)
        pltpu.sync_copy(dis_hbm, dis_v)
        pltpu.sync_copy(b2_hbm, b2_v)
        zeros16 = jnp.zeros((16,), jnp.float32)

        def zb(i, _):
            p_v[pl.ds(i * 16, 16)] = zeros16
            return 0

        lax.fori_loop(0, (2 * N_PAD) // 16, zb, 0)
        ones = jnp.ones((16,), jnp.int32)

        def body(j, _):
            for k in range(8):
                s16 = srcv[j, pl.ds(k * 16, 16)]
                d16 = dstv[j, pl.ds(k * 16, 16)]
                s2 = s16 + s16
                d2 = d16 + d16
                g0 = plsc.load_gather(u_v, [s2])
                g1 = plsc.load_gather(u_v, [s2 + ones])
                plsc.addupdate_scatter(p_v, [d2], g0)
                plsc.addupdate_scatter(p_v, [d2 + ones], g1)
            return 0

        lax.fori_loop(0, ROWS_C, body, 0)
        pltpu.sync_copy(p_v, sp_sh.at[sid])
        plsc.subcore_barrier()

        fbase = sid * 2 * NPT

        def zb2(i, _):
            o_v[pl.ds(i * 16, 16)] = zeros16
            return 0

        lax.fori_loop(0, (2 * NPT) // 16, zb2, 0)
        for t in range(NS):
            pltpu.sync_copy(sp_sh.at[t, pl.ds(fbase, 2 * NPT)], tmp_v)

            def ab(i, _):
                o_v[pl.ds(i * 16, 16)] += tmp_v[pl.ds(i * 16, 16)]
                return 0

            lax.fori_loop(0, (2 * NPT) // 16, ab, 0)

        b2v = b2_v[...]
        iota = lax.iota(jnp.int32, 16)
        half_iota = lax.shift_right_logical(iota, 1)
        nbase = sid * NPT

        def fb(i, _):
            pv = o_v[pl.ds(i * 16, 16)]
            uv = u_v[pl.ds(fbase + i * 16, 16)]
            dpair = plsc.load_gather(dis_v, [nbase + i * 8 + half_iota])
            o_v[pl.ds(i * 16, 16)] = dpair * (pv + uv) + b2v
            return 0

        lax.fori_loop(0, (2 * NPT) // 16, fb, 0)
        pltpu.sync_copy(o_v, out_hbm.at[pl.ds(fbase, 2 * NPT)])


_fin_call = pl.kernel(
    _fin_body,
    out_type=jax.ShapeDtypeStruct((2 * N_PAD,), jnp.float32),
    mesh=_mesh,
    scratch_types=[
        pltpu.VMEM((ROWS_C, 128), jnp.int32),
        pltpu.VMEM((ROWS_C, 128), jnp.int32),
        pltpu.VMEM((2 * N_PAD,), jnp.float32),
        pltpu.VMEM((2 * N_PAD,), jnp.float32),
        pltpu.VMEM((2 * NPT,), jnp.float32),
        pltpu.VMEM((2 * NPT,), jnp.float32),
        pltpu.VMEM((16,), jnp.float32),
        pltpu.VMEM((N_PAD,), jnp.float32),
        pltpu.VMEM_SHARED((NS, 2 * N_PAD), jnp.float32),
    ],
    compiler_params=_sc_params,
)


def kernel(x, edge_index, W1, b1, W2, b2):
    ei = edge_index.astype(jnp.int32)
    src, dst = ei[0], ei[1]
    padi = jnp.arange(E_PAD - E, dtype=jnp.int32)
    pad_src = padi % N_NODES
    pad_dst = N_NODES + padi % (N_PAD - N_NODES)
    srcp = jnp.concatenate([src, pad_src]).reshape(E_ROWS, 128)
    dstp = jnp.concatenate([dst, pad_dst]).reshape(E_ROWS, 128)

    degp = _deg_call(dstp)
    dis, y0, y1 = _prep_call(x, degp)
    agg0, agg1 = _agg_call(srcp, dstp, y0, y1)
    u = _main_call(dis, agg0, agg1, y0, y1, W1, b1.reshape(1, HIDDEN), W2)
    out_flat = _fin_call(srcp, dstp, u.reshape(2 * N_PAD),
                         dis.reshape(N_PAD), jnp.tile(b2, 8))
    return out_flat.reshape(N_PAD, OUT_CH)[:N_NODES]


# speedup vs baseline: 28.8565x; 1.0048x over previous
import jax
import jax.numpy as jnp
from jax import lax
from jax.experimental import pallas as pl
from jax.experimental.pallas import tpu as pltpu
from jax.experimental.pallas import tpu_sc as plsc

N_NODES = 10000
N_PAD = 10240
E = 160000
EC = 64
E_ROWS = 2560
E_PAD = E_ROWS * EC
IN_CH = 256
HALF = 128
HIDDEN = 512
OUT_CH = 2
NC = 2
NS = 16
NW = NC * NS
ROWS_A = E_ROWS // NW
ROWS_C = E_ROWS // NS
NPT = N_PAD // NS
RB = 2048
GRID = 5

_mesh = plsc.VectorSubcoreMesh(core_axis_name="c", subcore_axis_name="s")
_sc_params = pltpu.CompilerParams(needs_layout_passes=False)


def _deg_body(dst_hbm, out_hbm, idx_v, hist_v):
    cid = lax.axis_index("c")
    sid = lax.axis_index("s")
    wid = sid * NC + cid
    zeros16 = jnp.zeros((16,), jnp.float32)

    def zb(i, _):
        hist_v[pl.ds(i * 16, 16)] = zeros16
        return 0

    lax.fori_loop(0, N_PAD // 16, zb, 0)
    pltpu.sync_copy(dst_hbm.at[pl.ds(wid * ROWS_A, ROWS_A)], idx_v)
    ones16 = jnp.ones((16,), jnp.float32)

    def body(j, _):
        for k in range(EC // 16):
            idx = idx_v[j, pl.ds(k * 16, 16)]
            plsc.addupdate_scatter(hist_v, [idx], ones16)
        return 0

    lax.fori_loop(0, ROWS_A, body, 0)
    pltpu.sync_copy(hist_v, out_hbm.at[wid])


_deg_call = pl.kernel(
    _deg_body,
    out_type=jax.ShapeDtypeStruct((NW, N_PAD), jnp.float32),
    mesh=_mesh,
    scratch_types=[
        pltpu.VMEM((ROWS_A, EC), jnp.int32),
        pltpu.VMEM((N_PAD,), jnp.float32),
    ],
    compiler_params=_sc_params,
)


def _prep_body(x_ref, degp_ref, dis_ref, y0_ref, y1_ref):
    deg = jnp.sum(degp_ref[...], axis=0) + 1.0
    dis = lax.rsqrt(deg)[:, None]
    dis_ref[...] = dis
    xb = x_ref[...]
    y0_ref[...] = xb[:, :HALF] * dis
    y1_ref[...] = xb[:, HALF:] * dis


_prep_call = pl.pallas_call(
    _prep_body,
    grid=(GRID,),
    in_specs=[
        pl.BlockSpec((RB, IN_CH), lambda i: (i, 0)),
        pl.BlockSpec((NW, RB), lambda i: (0, i)),
    ],
    out_specs=[
        pl.BlockSpec((RB, 1), lambda i: (i, 0)),
        pl.BlockSpec((RB, HALF), lambda i: (i, 0)),
        pl.BlockSpec((RB, HALF), lambda i: (i, 0)),
    ],
    out_shape=[
        jax.ShapeDtypeStruct((N_PAD, 1), jnp.float32),
        jax.ShapeDtypeStruct((N_PAD, HALF), jnp.float32),
        jax.ShapeDtypeStruct((N_PAD, HALF), jnp.float32),
    ],
)


def _agg_body(src_hbm, dst_hbm, y0_hbm, y1_hbm, agg0_hbm, agg1_hbm,
              srcv, dstv, b0, b1, b2, b3, acc_sh,
              g0, g1, g2, g3, s0, s1, s2, s3):
    cid = lax.axis_index("c")
    sid = lax.axis_index("s")
    base = sid * ROWS_C
    hrows = ROWS_C // 4
    bufs = (b0, b1, b2, b3)
    gsems = (g0, g1, g2, g3)
    ssems = (s0, s1, s2, s3)

    zeros16 = jnp.zeros((16,), jnp.float32)

    def zb(i, _):
        for k in range(8):
            b0[i, pl.ds(k * 16, 16)] = zeros16
        return 0

    lax.fori_loop(0, EC, zb, 0)
    for k in range(NPT // EC):
        pltpu.sync_copy(b0, acc_sh.at[pl.ds(sid * NPT + k * EC, EC)])
    plsc.subcore_barrier()

    def do_core(y_hbm):
        for h in range(4):
            pltpu.sync_copy(src_hbm.at[pl.ds(base + h * hrows, hrows)], srcv)
            pltpu.sync_copy(dst_hbm.at[pl.ds(base + h * hrows, hrows)], dstv)
            pltpu.async_copy(y_hbm.at[srcv.at[0]], bufs[0], gsems[0])

            def body(ji, _):
                for b in range(4):
                    j = ji * 4 + b
                    nb_ = (b + 1) % 4

                    @pl.when(j >= 3)
                    def _():
                        pltpu.make_async_copy(
                            bufs[nb_], acc_sh.at[dstv.at[j]], ssems[nb_]
                        ).wait()

                    @pl.when(j + 1 < hrows)
                    def _():
                        pltpu.async_copy(
                            y_hbm.at[srcv.at[j + 1]], bufs[nb_], gsems[nb_])

                    pltpu.make_async_copy(
                        y_hbm.at[srcv.at[j]], bufs[b], gsems[b]).wait()
                    pltpu.async_copy(
                        bufs[b], acc_sh.at[dstv.at[j]], ssems[b], add=True)
                return 0

            lax.fori_loop(0, hrows // 4, body, 0)
            for b in (1, 2, 3):
                pltpu.make_async_copy(
                    bufs[b], acc_sh.at[dstv.at[0]], ssems[b]).wait()

    @pl.when(cid == 0)
    def _():
        do_core(y0_hbm)

    @pl.when(cid == 1)
    def _():
        do_core(y1_hbm)

    plsc.subcore_barrier()
    nb = sid * NPT

    @pl.when(cid == 0)
    def _():
        pltpu.sync_copy(acc_sh.at[pl.ds(nb, NPT)], agg0_hbm.at[pl.ds(nb, NPT)])

    @pl.when(cid == 1)
    def _():
        pltpu.sync_copy(acc_sh.at[pl.ds(nb, NPT)], agg1_hbm.at[pl.ds(nb, NPT)])


_agg_call = pl.kernel(
    _agg_body,
    out_type=(
        jax.ShapeDtypeStruct((N_PAD, HALF), jnp.float32),
        jax.ShapeDtypeStruct((N_PAD, HALF), jnp.float32),
    ),
    mesh=_mesh,
    scratch_types=[
        pltpu.VMEM((ROWS_C // 4, EC), jnp.int32),
        pltpu.VMEM((ROWS_C // 4, EC), jnp.int32),
        pltpu.VMEM((EC, HALF), jnp.float32),
        pltpu.VMEM((EC, HALF), jnp.float32),
        pltpu.VMEM((EC, HALF), jnp.float32),
        pltpu.VMEM((EC, HALF), jnp.float32),
        pltpu.VMEM_SHARED((N_PAD, HALF), jnp.float32),
        pltpu.SemaphoreType.DMA,
        pltpu.SemaphoreType.DMA,
        pltpu.SemaphoreType.DMA,
        pltpu.SemaphoreType.DMA,
        pltpu.SemaphoreType.DMA,
        pltpu.SemaphoreType.DMA,
        pltpu.SemaphoreType.DMA,
        pltpu.SemaphoreType.DMA,
    ],
    compiler_params=_sc_params,
)


def _main_body(dis_ref, agg0_ref, agg1_ref, y0_ref, y1_ref,
               w1_ref, b1_ref, w2_ref, u_ref):
    dis = dis_ref[...]
    z0 = (agg0_ref[...] + y0_ref[...]) * dis
    z1 = (agg1_ref[...] + y1_ref[...]) * dis
    w1 = w1_ref[...]
    h = jnp.dot(z0, w1[:HALF], preferred_element_type=jnp.float32)
    h = h + jnp.dot(z1, w1[HALF:], preferred_element_type=jnp.float32)
    h = jnp.maximum(h + b1_ref[...], 0.0)
    u_ref[...] = jnp.dot(h, w2_ref[...], preferred_element_type=jnp.float32) * dis


_main_call = pl.pallas_call(
    _main_body,
    grid=(GRID,),
    in_specs=[
        pl.BlockSpec((RB, 1), lambda i: (i, 0)),
        pl.BlockSpec((RB, HALF), lambda i: (i, 0)),
        pl.BlockSpec((RB, HALF), lambda i: (i, 0)),
        pl.BlockSpec((RB, HALF), lambda i: (i, 0)),
        pl.BlockSpec((RB, HALF), lambda i: (i, 0)),
        pl.BlockSpec((IN_CH, HIDDEN), lambda i: (0, 0)),
        pl.BlockSpec((1, HIDDEN), lambda i: (0, 0)),
        pl.BlockSpec((HIDDEN, OUT_CH), lambda i: (0, 0)),
    ],
    out_specs=pl.BlockSpec((RB, OUT_CH), lambda i: (i, 0)),
    out_shape=jax.ShapeDtypeStruct((N_PAD, OUT_CH), jnp.float32),
)


def _fin_body(src_hbm, dst_hbm, u_hbm, dis_hbm, b2_hbm, out_hbm,
              srcv, dstv, u_v, p_v, tmp_v, o_v, b2_v, dis_v, sp_sh):
    cid = lax.axis_index("c")
    sid = lax.axis_index("s")

    @pl.when(cid == 0)
    def _():
        pltpu.sync_copy(src_hbm.at[pl.ds(sid * ROWS_C, ROWS_C)], srcv)
        pltpu.sync_copy(dst_hbm.at[pl.ds(sid * ROWS_C, ROWS_C)], dstv)
        pltpu.sync_copy(u_hbm, u_v)
        pltpu.sync_copy(dis_hbm, dis_v)
        pltpu.sync_copy(b2_hbm, b2_v)
        zeros16 = jnp.zeros((16,), jnp.float32)

        def zb(i, _):
            p_v[pl.ds(i * 16, 16)] = zeros16
            return 0

        lax.fori_loop(0, (2 * N_PAD) // 16, zb, 0)
        ones = jnp.ones((16,), jnp.int32)

        def body(j, _):
            for k in range(EC // 16):
                s16 = srcv[j, pl.ds(k * 16, 16)]
                d16 = dstv[j, pl.ds(k * 16, 16)]
                s2 = s16 + s16
                d2 = d16 + d16
                g0 = plsc.load_gather(u_v, [s2])
                g1 = plsc.load_gather(u_v, [s2 + ones])
                plsc.addupdate_scatter(p_v, [d2], g0)
                plsc.addupdate_scatter(p_v, [d2 + ones], g1)
            return 0

        lax.fori_loop(0, ROWS_C, body, 0)
        pltpu.sync_copy(p_v, sp_sh.at[sid])
        plsc.subcore_barrier()

        fbase = sid * 2 * NPT

        def zb2(i, _):
            o_v[pl.ds(i * 16, 16)] = zeros16
            return 0

        lax.fori_loop(0, (2 * NPT) // 16, zb2, 0)
        for t in range(NS):
            pltpu.sync_copy(sp_sh.at[t, pl.ds(fbase, 2 * NPT)], tmp_v)

            def ab(i, _):
                o_v[pl.ds(i * 16, 16)] += tmp_v[pl.ds(i * 16, 16)]
                return 0

            lax.fori_loop(0, (2 * NPT) // 16, ab, 0)

        b2v = b2_v[...]
        iota = lax.iota(jnp.int32, 16)
        half_iota = lax.shift_right_logical(iota, 1)
        nbase = sid * NPT

        def fb(i, _):
            pv = o_v[pl.ds(i * 16, 16)]
            uv = u_v[pl.ds(fbase + i * 16, 16)]
            dpair = plsc.load_gather(dis_v, [nbase + i * 8 + half_iota])
            o_v[pl.ds(i * 16, 16)] = dpair * (pv + uv) + b2v
            return 0

        lax.fori_loop(0, (2 * NPT) // 16, fb, 0)
        pltpu.sync_copy(o_v, out_hbm.at[pl.ds(fbase, 2 * NPT)])


_fin_call = pl.kernel(
    _fin_body,
    out_type=jax.ShapeDtypeStruct((2 * N_PAD,), jnp.float32),
    mesh=_mesh,
    scratch_types=[
        pltpu.VMEM((ROWS_C, EC), jnp.int32),
        pltpu.VMEM((ROWS_C, EC), jnp.int32),
        pltpu.VMEM((2 * N_PAD,), jnp.float32),
        pltpu.VMEM((2 * N_PAD,), jnp.float32),
        pltpu.VMEM((2 * NPT,), jnp.float32),
        pltpu.VMEM((2 * NPT,), jnp.float32),
        pltpu.VMEM((16,), jnp.float32),
        pltpu.VMEM((N_PAD,), jnp.float32),
        pltpu.VMEM_SHARED((NS, 2 * N_PAD), jnp.float32),
    ],
    compiler_params=_sc_params,
)


def kernel(x, edge_index, W1, b1, W2, b2):
    ei = edge_index.astype(jnp.int32)
    src, dst = ei[0], ei[1]
    padi = jnp.arange(E_PAD - E, dtype=jnp.int32)
    pad_src = padi % N_NODES
    pad_dst = N_NODES + padi % (N_PAD - N_NODES)
    srcp = jnp.concatenate([src, pad_src]).reshape(E_ROWS, EC)
    dstp = jnp.concatenate([dst, pad_dst]).reshape(E_ROWS, EC)

    degp = _deg_call(dstp)
    dis, y0, y1 = _prep_call(x, degp)
    agg0, agg1 = _agg_call(srcp, dstp, y0, y1)
    u = _main_call(dis, agg0, agg1, y0, y1, W1, b1.reshape(1, HIDDEN), W2)
    out_flat = _fin_call(srcp, dstp, u.reshape(2 * N_PAD),
                         dis.reshape(N_PAD), jnp.tile(b2, 8))
    return out_flat.reshape(N_PAD, OUT_CH)[:N_NODES]


# speedup vs baseline: 30.7913x; 1.0670x over previous
import jax
import jax.numpy as jnp
from jax import lax
from jax.experimental import pallas as pl
from jax.experimental.pallas import tpu as pltpu
from jax.experimental.pallas import tpu_sc as plsc

N_NODES = 10000
N_PAD = 10240
E = 160000
EC = 64
E_ROWS = 2560
E_PAD = E_ROWS * EC
IN_CH = 256
HALF = 128
HIDDEN = 512
OUT_CH = 2
NC = 2
NS = 16
NW = NC * NS
ROWS_A = E_ROWS // NW
ROWS_C = E_ROWS // NS
NPT = N_PAD // NS
RB = 2048
GRID = 5

_mesh = plsc.VectorSubcoreMesh(core_axis_name="c", subcore_axis_name="s")
_sc_params = pltpu.CompilerParams(needs_layout_passes=False)


def _deg_body(dst_hbm, out_hbm, idx_v, hist_v):
    cid = lax.axis_index("c")
    sid = lax.axis_index("s")
    wid = sid * NC + cid
    zeros16 = jnp.zeros((16,), jnp.float32)

    def zb(i, _):
        for k in range(8):
            hist_v[pl.ds(i * 128 + k * 16, 16)] = zeros16
        return 0

    lax.fori_loop(0, N_PAD // 128, zb, 0)
    pltpu.sync_copy(dst_hbm.at[pl.ds(wid * ROWS_A, ROWS_A)], idx_v)
    ones16 = jnp.ones((16,), jnp.float32)

    def body(j, _):
        for k in range(EC // 16):
            idx = idx_v[j, pl.ds(k * 16, 16)]
            plsc.addupdate_scatter(hist_v, [idx], ones16)
        return 0

    lax.fori_loop(0, ROWS_A, body, 0)
    pltpu.sync_copy(hist_v, out_hbm.at[wid])


_deg_call = pl.kernel(
    _deg_body,
    out_type=jax.ShapeDtypeStruct((NW, N_PAD), jnp.float32),
    mesh=_mesh,
    scratch_types=[
        pltpu.VMEM((ROWS_A, EC), jnp.int32),
        pltpu.VMEM((N_PAD,), jnp.float32),
    ],
    compiler_params=_sc_params,
)


def _prep_body(x_ref, degp_ref, dis_ref, y0_ref, y1_ref):
    deg = jnp.sum(degp_ref[...], axis=0) + 1.0
    dis = lax.rsqrt(deg)[:, None]
    dis_ref[...] = dis
    xb = x_ref[...]
    y0_ref[...] = xb[:, :HALF] * dis
    y1_ref[...] = xb[:, HALF:] * dis


_prep_call = pl.pallas_call(
    _prep_body,
    grid=(GRID,),
    in_specs=[
        pl.BlockSpec((RB, IN_CH), lambda i: (i, 0)),
        pl.BlockSpec((NW, RB), lambda i: (0, i)),
    ],
    out_specs=[
        pl.BlockSpec((RB, 1), lambda i: (i, 0)),
        pl.BlockSpec((RB, HALF), lambda i: (i, 0)),
        pl.BlockSpec((RB, HALF), lambda i: (i, 0)),
    ],
    out_shape=[
        jax.ShapeDtypeStruct((N_PAD, 1), jnp.float32),
        jax.ShapeDtypeStruct((N_PAD, HALF), jnp.float32),
        jax.ShapeDtypeStruct((N_PAD, HALF), jnp.float32),
    ],
)


def _agg_body(src_hbm, dst_hbm, y0_hbm, y1_hbm, agg0_hbm, agg1_hbm,
              srcv, dstv, b0, b1, b2, b3, acc_sh,
              g0, g1, g2, g3, s0, s1, s2, s3):
    cid = lax.axis_index("c")
    sid = lax.axis_index("s")
    base = sid * ROWS_C
    hrows = ROWS_C // 4
    bufs = (b0, b1, b2, b3)
    gsems = (g0, g1, g2, g3)
    ssems = (s0, s1, s2, s3)

    zeros16 = jnp.zeros((16,), jnp.float32)

    def zb(i, _):
        for k in range(HALF // 16):
            b0[i, pl.ds(k * 16, 16)] = zeros16
        return 0

    lax.fori_loop(0, EC, zb, 0)
    for k in range(NPT // EC):
        pltpu.async_copy(b0, acc_sh.at[pl.ds(sid * NPT + k * EC, EC)], g0)
    for k in range(NPT // EC):
        pltpu.make_async_copy(b0, acc_sh.at[pl.ds(sid * NPT, EC)], g0).wait()
    plsc.subcore_barrier()

    def do_core(y_hbm):
        for h in range(4):
            pltpu.sync_copy(src_hbm.at[pl.ds(base + h * hrows, hrows)], srcv)
            pltpu.sync_copy(dst_hbm.at[pl.ds(base + h * hrows, hrows)], dstv)
            pltpu.async_copy(y_hbm.at[srcv.at[0]], bufs[0], gsems[0])

            def body(ji, _):
                for b in range(4):
                    j = ji * 4 + b
                    nb_ = (b + 1) % 4

                    @pl.when(j >= 3)
                    def _():
                        pltpu.make_async_copy(
                            bufs[nb_], acc_sh.at[dstv.at[j]], ssems[nb_]
                        ).wait()

                    @pl.when(j + 1 < hrows)
                    def _():
                        pltpu.async_copy(
                            y_hbm.at[srcv.at[j + 1]], bufs[nb_], gsems[nb_])

                    pltpu.make_async_copy(
                        y_hbm.at[srcv.at[j]], bufs[b], gsems[b]).wait()
                    pltpu.async_copy(
                        bufs[b], acc_sh.at[dstv.at[j]], ssems[b], add=True)
                return 0

            lax.fori_loop(0, hrows // 4, body, 0)
            for b in (1, 2, 3):
                pltpu.make_async_copy(
                    bufs[b], acc_sh.at[dstv.at[0]], ssems[b]).wait()

    @pl.when(cid == 0)
    def _():
        do_core(y0_hbm)

    @pl.when(cid == 1)
    def _():
        do_core(y1_hbm)

    plsc.subcore_barrier()
    nb = sid * NPT

    @pl.when(cid == 0)
    def _():
        pltpu.sync_copy(acc_sh.at[pl.ds(nb, NPT)], agg0_hbm.at[pl.ds(nb, NPT)])

    @pl.when(cid == 1)
    def _():
        pltpu.sync_copy(acc_sh.at[pl.ds(nb, NPT)], agg1_hbm.at[pl.ds(nb, NPT)])


_agg_call = pl.kernel(
    _agg_body,
    out_type=(
        jax.ShapeDtypeStruct((N_PAD, HALF), jnp.float32),
        jax.ShapeDtypeStruct((N_PAD, HALF), jnp.float32),
    ),
    mesh=_mesh,
    scratch_types=[
        pltpu.VMEM((ROWS_C // 4, EC), jnp.int32),
        pltpu.VMEM((ROWS_C // 4, EC), jnp.int32),
        pltpu.VMEM((EC, HALF), jnp.float32),
        pltpu.VMEM((EC, HALF), jnp.float32),
        pltpu.VMEM((EC, HALF), jnp.float32),
        pltpu.VMEM((EC, HALF), jnp.float32),
        pltpu.VMEM_SHARED((N_PAD, HALF), jnp.float32),
        pltpu.SemaphoreType.DMA,
        pltpu.SemaphoreType.DMA,
        pltpu.SemaphoreType.DMA,
        pltpu.SemaphoreType.DMA,
        pltpu.SemaphoreType.DMA,
        pltpu.SemaphoreType.DMA,
        pltpu.SemaphoreType.DMA,
        pltpu.SemaphoreType.DMA,
    ],
    compiler_params=_sc_params,
)


def _main_body(dis_ref, agg0_ref, agg1_ref, y0_ref, y1_ref,
               w1_ref, b1_ref, w2_ref, u_ref):
    dis = dis_ref[...]
    z0 = (agg0_ref[...] + y0_ref[...]) * dis
    z1 = (agg1_ref[...] + y1_ref[...]) * dis
    w1 = w1_ref[...]
    h = jnp.dot(z0, w1[:HALF], preferred_element_type=jnp.float32)
    h = h + jnp.dot(z1, w1[HALF:], preferred_element_type=jnp.float32)
    h = jnp.maximum(h + b1_ref[...], 0.0)
    u_ref[...] = jnp.dot(h, w2_ref[...], preferred_element_type=jnp.float32) * dis


_main_call = pl.pallas_call(
    _main_body,
    grid=(GRID,),
    in_specs=[
        pl.BlockSpec((RB, 1), lambda i: (i, 0)),
        pl.BlockSpec((RB, HALF), lambda i: (i, 0)),
        pl.BlockSpec((RB, HALF), lambda i: (i, 0)),
        pl.BlockSpec((RB, HALF), lambda i: (i, 0)),
        pl.BlockSpec((RB, HALF), lambda i: (i, 0)),
        pl.BlockSpec((IN_CH, HIDDEN), lambda i: (0, 0)),
        pl.BlockSpec((1, HIDDEN), lambda i: (0, 0)),
        pl.BlockSpec((HIDDEN, OUT_CH), lambda i: (0, 0)),
    ],
    out_specs=pl.BlockSpec((RB, OUT_CH), lambda i: (i, 0)),
    out_shape=jax.ShapeDtypeStruct((N_PAD, OUT_CH), jnp.float32),
)


def _fin_body(src_hbm, dst_hbm, u_hbm, dis_hbm, b2_hbm, out_hbm,
              srcv, dstv, u_v, p_v, tmp_v, o_v, b2_v, dis_v, sp_sh, sg0):
    cid = lax.axis_index("c")
    sid = lax.axis_index("s")

    @pl.when(cid == 0)
    def _():
        pltpu.async_copy(src_hbm.at[pl.ds(sid * ROWS_C, ROWS_C)], srcv, sg0)
        pltpu.async_copy(dst_hbm.at[pl.ds(sid * ROWS_C, ROWS_C)], dstv, sg0)
        pltpu.async_copy(u_hbm, u_v, sg0)
        pltpu.async_copy(dis_hbm, dis_v, sg0)
        pltpu.async_copy(b2_hbm, b2_v, sg0)
        zeros16 = jnp.zeros((16,), jnp.float32)

        def zb(i, _):
            for k in range(8):
                p_v[pl.ds(i * 128 + k * 16, 16)] = zeros16
            return 0

        lax.fori_loop(0, (2 * N_PAD) // 128, zb, 0)
        pltpu.make_async_copy(src_hbm.at[pl.ds(sid * ROWS_C, ROWS_C)], srcv, sg0).wait()
        pltpu.make_async_copy(dst_hbm.at[pl.ds(sid * ROWS_C, ROWS_C)], dstv, sg0).wait()
        pltpu.make_async_copy(u_hbm, u_v, sg0).wait()
        pltpu.make_async_copy(dis_hbm, dis_v, sg0).wait()
        pltpu.make_async_copy(b2_hbm, b2_v, sg0).wait()
        ones = jnp.ones((16,), jnp.int32)

        def body(j, _):
            for k in range(EC // 16):
                s16 = srcv[j, pl.ds(k * 16, 16)]
                d16 = dstv[j, pl.ds(k * 16, 16)]
                s2 = s16 + s16
                d2 = d16 + d16
                g0 = plsc.load_gather(u_v, [s2])
                g1 = plsc.load_gather(u_v, [s2 + ones])
                plsc.addupdate_scatter(p_v, [d2], g0)
                plsc.addupdate_scatter(p_v, [d2 + ones], g1)
            return 0

        lax.fori_loop(0, ROWS_C, body, 0)
        pltpu.sync_copy(p_v, sp_sh.at[sid])
        plsc.subcore_barrier()

        fbase = sid * 2 * NPT

        def zb2(i, _):
            for k in range(8):
                o_v[pl.ds(i * 128 + k * 16, 16)] = zeros16
            return 0

        lax.fori_loop(0, (2 * NPT) // 128, zb2, 0)
        for t in range(NS):
            pltpu.sync_copy(sp_sh.at[t, pl.ds(fbase, 2 * NPT)], tmp_v)

            def ab(i, _):
                for k in range(8):
                    o_v[pl.ds(i * 128 + k * 16, 16)] += tmp_v[pl.ds(i * 128 + k * 16, 16)]
                return 0

            lax.fori_loop(0, (2 * NPT) // 128, ab, 0)

        b2v = b2_v[...]
        iota = lax.iota(jnp.int32, 16)
        half_iota = lax.shift_right_logical(iota, 1)
        nbase = sid * NPT

        def fb(i, _):
            pv = o_v[pl.ds(i * 16, 16)]
            uv = u_v[pl.ds(fbase + i * 16, 16)]
            dpair = plsc.load_gather(dis_v, [nbase + i * 8 + half_iota])
            o_v[pl.ds(i * 16, 16)] = dpair * (pv + uv) + b2v
            return 0

        lax.fori_loop(0, (2 * NPT) // 16, fb, 0)
        pltpu.sync_copy(o_v, out_hbm.at[pl.ds(fbase, 2 * NPT)])


_fin_call = pl.kernel(
    _fin_body,
    out_type=jax.ShapeDtypeStruct((2 * N_PAD,), jnp.float32),
    mesh=_mesh,
    scratch_types=[
        pltpu.VMEM((ROWS_C, EC), jnp.int32),
        pltpu.VMEM((ROWS_C, EC), jnp.int32),
        pltpu.VMEM((2 * N_PAD,), jnp.float32),
        pltpu.VMEM((2 * N_PAD,), jnp.float32),
        pltpu.VMEM((2 * NPT,), jnp.float32),
        pltpu.VMEM((2 * NPT,), jnp.float32),
        pltpu.VMEM((16,), jnp.float32),
        pltpu.VMEM((N_PAD,), jnp.float32),
        pltpu.VMEM_SHARED((NS, 2 * N_PAD), jnp.float32),
        pltpu.SemaphoreType.DMA,
    ],
    compiler_params=_sc_params,
)


def kernel(x, edge_index, W1, b1, W2, b2):
    ei = edge_index.astype(jnp.int32)
    src, dst = ei[0], ei[1]
    padi = jnp.arange(E_PAD - E, dtype=jnp.int32)
    pad_src = padi % N_NODES
    pad_dst = N_NODES + padi % (N_PAD - N_NODES)
    srcp = jnp.concatenate([src, pad_src]).reshape(E_ROWS, EC)
    dstp = jnp.concatenate([dst, pad_dst]).reshape(E_ROWS, EC)

    degp = _deg_call(dstp)
    dis, y0, y1 = _prep_call(x, degp)
    agg0, agg1 = _agg_call(srcp, dstp, y0, y1)
    u = _main_call(dis, agg0, agg1, y0, y1, W1, b1.reshape(1, HIDDEN), W2)
    out_flat = _fin_call(srcp, dstp, u.reshape(2 * N_PAD),
                         dis.reshape(N_PAD), jnp.tile(b2, 8))
    return out_flat.reshape(N_PAD, OUT_CH)[:N_NODES]


# speedup vs baseline: 30.8411x; 1.0016x over previous
import jax
import jax.numpy as jnp
from jax import lax
from jax.experimental import pallas as pl
from jax.experimental.pallas import tpu as pltpu
from jax.experimental.pallas import tpu_sc as plsc

N_NODES = 10000
N_PAD = 10240
E = 160000
EC = 64
E_ROWS = 2560
E_PAD = E_ROWS * EC
IN_CH = 256
HALF = 128
HIDDEN = 512
OUT_CH = 2
NC = 2
NS = 16
NW = NC * NS
ROWS_A = E_ROWS // NW
ROWS_C = E_ROWS // NS
NPT = N_PAD // NS
RB = 2048
GRID = 5

_mesh = plsc.VectorSubcoreMesh(core_axis_name="c", subcore_axis_name="s")
_sc_params = pltpu.CompilerParams(needs_layout_passes=False)


def _deg_body(dst_hbm, out_hbm, idx_v, hist_v):
    cid = lax.axis_index("c")
    sid = lax.axis_index("s")
    wid = sid * NC + cid
    zeros16 = jnp.zeros((16,), jnp.float32)

    def zb(i, _):
        for k in range(8):
            hist_v[pl.ds(i * 128 + k * 16, 16)] = zeros16
        return 0

    lax.fori_loop(0, N_PAD // 128, zb, 0)
    pltpu.sync_copy(dst_hbm.at[pl.ds(wid * ROWS_A, ROWS_A)], idx_v)
    ones16 = jnp.ones((16,), jnp.float32)

    def body(j, _):
        for k in range(EC // 16):
            idx = idx_v[j, pl.ds(k * 16, 16)]
            plsc.addupdate_scatter(hist_v, [idx], ones16)
        return 0

    lax.fori_loop(0, ROWS_A, body, 0)
    pltpu.sync_copy(hist_v, out_hbm.at[wid])


_deg_call = pl.kernel(
    _deg_body,
    out_type=jax.ShapeDtypeStruct((NW, N_PAD), jnp.float32),
    mesh=_mesh,
    scratch_types=[
        pltpu.VMEM((ROWS_A, EC), jnp.int32),
        pltpu.VMEM((N_PAD,), jnp.float32),
    ],
    compiler_params=_sc_params,
)


def _prep_body(x_ref, degp_ref, dis_ref, y0_ref, y1_ref):
    deg = jnp.sum(degp_ref[...], axis=0) + 1.0
    dis = lax.rsqrt(deg)[:, None]
    dis_ref[...] = dis
    xb = x_ref[...]
    y0_ref[...] = xb[:, :HALF] * dis
    y1_ref[...] = xb[:, HALF:] * dis


_prep_call = pl.pallas_call(
    _prep_body,
    grid=(GRID,),
    in_specs=[
        pl.BlockSpec((RB, IN_CH), lambda i: (i, 0)),
        pl.BlockSpec((NW, RB), lambda i: (0, i)),
    ],
    out_specs=[
        pl.BlockSpec((RB, 1), lambda i: (i, 0)),
        pl.BlockSpec((RB, HALF), lambda i: (i, 0)),
        pl.BlockSpec((RB, HALF), lambda i: (i, 0)),
    ],
    out_shape=[
        jax.ShapeDtypeStruct((N_PAD, 1), jnp.float32),
        jax.ShapeDtypeStruct((N_PAD, HALF), jnp.float32),
        jax.ShapeDtypeStruct((N_PAD, HALF), jnp.float32),
    ],
)


def _agg_body(src_hbm, dst_hbm, y0_hbm, y1_hbm, agg0_hbm, agg1_hbm,
              srcv, dstv, b0, b1, b2, b3, acc_sh,
              g0, g1, g2, g3, s0, s1, s2, s3):
    cid = lax.axis_index("c")
    sid = lax.axis_index("s")
    base = sid * ROWS_C
    hrows = ROWS_C // 4
    bufs = (b0, b1, b2, b3)
    gsems = (g0, g1, g2, g3)
    ssems = (s0, s1, s2, s3)

    zeros16 = jnp.zeros((16,), jnp.float32)

    def zb(i, _):
        for k in range(HALF // 16):
            b0[i, pl.ds(k * 16, 16)] = zeros16
        return 0

    lax.fori_loop(0, EC, zb, 0)
    for k in range(NPT // EC):
        pltpu.async_copy(b0, acc_sh.at[pl.ds(sid * NPT + k * EC, EC)], g0)
    for k in range(NPT // EC):
        pltpu.make_async_copy(b0, acc_sh.at[pl.ds(sid * NPT, EC)], g0).wait()
    plsc.subcore_barrier()

    def do_core(y_hbm):
        for h in range(4):
            pltpu.sync_copy(src_hbm.at[pl.ds(base + h * hrows, hrows)], srcv)
            pltpu.sync_copy(dst_hbm.at[pl.ds(base + h * hrows, hrows)], dstv)
            pltpu.async_copy(y_hbm.at[srcv.at[0]], bufs[0], gsems[0])

            def body(ji, _):
                for b in range(4):
                    j = ji * 4 + b
                    nb_ = (b + 1) % 4

                    @pl.when(j >= 3)
                    def _():
                        pltpu.make_async_copy(
                            bufs[nb_], acc_sh.at[dstv.at[j]], ssems[nb_]
                        ).wait()

                    @pl.when(j + 1 < hrows)
                    def _():
                        pltpu.async_copy(
                            y_hbm.at[srcv.at[j + 1]], bufs[nb_], gsems[nb_])

                    pltpu.make_async_copy(
                        y_hbm.at[srcv.at[j]], bufs[b], gsems[b]).wait()
                    pltpu.async_copy(
                        bufs[b], acc_sh.at[dstv.at[j]], ssems[b], add=True)
                return 0

            lax.fori_loop(0, hrows // 4, body, 0)
            for b in (1, 2, 3):
                pltpu.make_async_copy(
                    bufs[b], acc_sh.at[dstv.at[0]], ssems[b]).wait()

    @pl.when(cid == 0)
    def _():
        do_core(y0_hbm)

    @pl.when(cid == 1)
    def _():
        do_core(y1_hbm)

    plsc.subcore_barrier()
    nb = sid * NPT

    @pl.when(cid == 0)
    def _():
        pltpu.sync_copy(acc_sh.at[pl.ds(nb, NPT)], agg0_hbm.at[pl.ds(nb, NPT)])

    @pl.when(cid == 1)
    def _():
        pltpu.sync_copy(acc_sh.at[pl.ds(nb, NPT)], agg1_hbm.at[pl.ds(nb, NPT)])


_agg_call = pl.kernel(
    _agg_body,
    out_type=(
        jax.ShapeDtypeStruct((N_PAD, HALF), jnp.float32),
        jax.ShapeDtypeStruct((N_PAD, HALF), jnp.float32),
    ),
    mesh=_mesh,
    scratch_types=[
        pltpu.VMEM((ROWS_C // 4, EC), jnp.int32),
        pltpu.VMEM((ROWS_C // 4, EC), jnp.int32),
        pltpu.VMEM((EC, HALF), jnp.float32),
        pltpu.VMEM((EC, HALF), jnp.float32),
        pltpu.VMEM((EC, HALF), jnp.float32),
        pltpu.VMEM((EC, HALF), jnp.float32),
        pltpu.VMEM_SHARED((N_PAD, HALF), jnp.float32),
        pltpu.SemaphoreType.DMA,
        pltpu.SemaphoreType.DMA,
        pltpu.SemaphoreType.DMA,
        pltpu.SemaphoreType.DMA,
        pltpu.SemaphoreType.DMA,
        pltpu.SemaphoreType.DMA,
        pltpu.SemaphoreType.DMA,
        pltpu.SemaphoreType.DMA,
    ],
    compiler_params=_sc_params,
)


def _main_body(dis_ref, agg0_ref, agg1_ref, y0_ref, y1_ref,
               w1_ref, b1_ref, w2_ref, u_ref):
    dis = dis_ref[...]
    z0 = ((agg0_ref[...] + y0_ref[...]) * dis).astype(jnp.bfloat16)
    z1 = ((agg1_ref[...] + y1_ref[...]) * dis).astype(jnp.bfloat16)
    w1 = w1_ref[...].astype(jnp.bfloat16)
    h = jnp.dot(z0, w1[:HALF], preferred_element_type=jnp.float32)
    h = h + jnp.dot(z1, w1[HALF:], preferred_element_type=jnp.float32)
    h = jnp.maximum(h + b1_ref[...], 0.0).astype(jnp.bfloat16)
    w2 = w2_ref[...].astype(jnp.bfloat16)
    u_ref[...] = jnp.dot(h, w2, preferred_element_type=jnp.float32) * dis


_main_call = pl.pallas_call(
    _main_body,
    grid=(GRID,),
    in_specs=[
        pl.BlockSpec((RB, 1), lambda i: (i, 0)),
        pl.BlockSpec((RB, HALF), lambda i: (i, 0)),
        pl.BlockSpec((RB, HALF), lambda i: (i, 0)),
        pl.BlockSpec((RB, HALF), lambda i: (i, 0)),
        pl.BlockSpec((RB, HALF), lambda i: (i, 0)),
        pl.BlockSpec((IN_CH, HIDDEN), lambda i: (0, 0)),
        pl.BlockSpec((1, HIDDEN), lambda i: (0, 0)),
        pl.BlockSpec((HIDDEN, OUT_CH), lambda i: (0, 0)),
    ],
    out_specs=pl.BlockSpec((RB, OUT_CH), lambda i: (i, 0)),
    out_shape=jax.ShapeDtypeStruct((N_PAD, OUT_CH), jnp.float32),
)


def _fin_body(src_hbm, dst_hbm, u_hbm, dis_hbm, b2_hbm, out_hbm,
              srcv, dstv, u_v, p_v, tmp_v, o_v, b2_v, dis_v, sp_sh, sg0):
    cid = lax.axis_index("c")
    sid = lax.axis_index("s")

    @pl.when(cid == 0)
    def _():
        pltpu.async_copy(src_hbm.at[pl.ds(sid * ROWS_C, ROWS_C)], srcv, sg0)
        pltpu.async_copy(dst_hbm.at[pl.ds(sid * ROWS_C, ROWS_C)], dstv, sg0)
        pltpu.async_copy(u_hbm, u_v, sg0)
        pltpu.async_copy(dis_hbm, dis_v, sg0)
        pltpu.async_copy(b2_hbm, b2_v, sg0)
        zeros16 = jnp.zeros((16,), jnp.float32)

        def zb(i, _):
            for k in range(8):
                p_v[pl.ds(i * 128 + k * 16, 16)] = zeros16
            return 0

        lax.fori_loop(0, (2 * N_PAD) // 128, zb, 0)
        pltpu.make_async_copy(src_hbm.at[pl.ds(sid * ROWS_C, ROWS_C)], srcv, sg0).wait()
        pltpu.make_async_copy(dst_hbm.at[pl.ds(sid * ROWS_C, ROWS_C)], dstv, sg0).wait()
        pltpu.make_async_copy(u_hbm, u_v, sg0).wait()
        pltpu.make_async_copy(dis_hbm, dis_v, sg0).wait()
        pltpu.make_async_copy(b2_hbm, b2_v, sg0).wait()
        ones = jnp.ones((16,), jnp.int32)

        def body(j, _):
            for k in range(EC // 16):
                s16 = srcv[j, pl.ds(k * 16, 16)]
                d16 = dstv[j, pl.ds(k * 16, 16)]
                s2 = s16 + s16
                d2 = d16 + d16
                g0 = plsc.load_gather(u_v, [s2])
                g1 = plsc.load_gather(u_v, [s2 + ones])
                plsc.addupdate_scatter(p_v, [d2], g0)
                plsc.addupdate_scatter(p_v, [d2 + ones], g1)
            return 0

        lax.fori_loop(0, ROWS_C, body, 0)
        pltpu.sync_copy(p_v, sp_sh.at[sid])
        plsc.subcore_barrier()

        fbase = sid * 2 * NPT

        def zb2(i, _):
            for k in range(8):
                o_v[pl.ds(i * 128 + k * 16, 16)] = zeros16
            return 0

        lax.fori_loop(0, (2 * NPT) // 128, zb2, 0)
        for t in range(NS):
            pltpu.sync_copy(sp_sh.at[t, pl.ds(fbase, 2 * NPT)], tmp_v)

            def ab(i, _):
                for k in range(8):
                    o_v[pl.ds(i * 128 + k * 16, 16)] += tmp_v[pl.ds(i * 128 + k * 16, 16)]
                return 0

            lax.fori_loop(0, (2 * NPT) // 128, ab, 0)

        b2v = b2_v[...]
        iota = lax.iota(jnp.int32, 16)
        half_iota = lax.shift_right_logical(iota, 1)
        nbase = sid * NPT

        def fb(i, _):
            pv = o_v[pl.ds(i * 16, 16)]
            uv = u_v[pl.ds(fbase + i * 16, 16)]
            dpair = plsc.load_gather(dis_v, [nbase + i * 8 + half_iota])
            o_v[pl.ds(i * 16, 16)] = dpair * (pv + uv) + b2v
            return 0

        lax.fori_loop(0, (2 * NPT) // 16, fb, 0)
        pltpu.sync_copy(o_v, out_hbm.at[pl.ds(fbase, 2 * NPT)])


_fin_call = pl.kernel(
    _fin_body,
    out_type=jax.ShapeDtypeStruct((2 * N_PAD,), jnp.float32),
    mesh=_mesh,
    scratch_types=[
        pltpu.VMEM((ROWS_C, EC), jnp.int32),
        pltpu.VMEM((ROWS_C, EC), jnp.int32),
        pltpu.VMEM((2 * N_PAD,), jnp.float32),
        pltpu.VMEM((2 * N_PAD,), jnp.float32),
        pltpu.VMEM((2 * NPT,), jnp.float32),
        pltpu.VMEM((2 * NPT,), jnp.float32),
        pltpu.VMEM((16,), jnp.float32),
        pltpu.VMEM((N_PAD,), jnp.float32),
        pltpu.VMEM_SHARED((NS, 2 * N_PAD), jnp.float32),
        pltpu.SemaphoreType.DMA,
    ],
    compiler_params=_sc_params,
)


def kernel(x, edge_index, W1, b1, W2, b2):
    ei = edge_index.astype(jnp.int32)
    src, dst = ei[0], ei[1]
    padi = jnp.arange(E_PAD - E, dtype=jnp.int32)
    pad_src = padi % N_NODES
    pad_dst = N_NODES + padi % (N_PAD - N_NODES)
    srcp = jnp.concatenate([src, pad_src]).reshape(E_ROWS, EC)
    dstp = jnp.concatenate([dst, pad_dst]).reshape(E_ROWS, EC)

    degp = _deg_call(dstp)
    dis, y0, y1 = _prep_call(x, degp)
    agg0, agg1 = _agg_call(srcp, dstp, y0, y1)
    u = _main_call(dis, agg0, agg1, y0, y1, W1, b1.reshape(1, HIDDEN), W2)
    out_flat = _fin_call(srcp, dstp, u.reshape(2 * N_PAD),
                         dis.reshape(N_PAD), jnp.tile(b2, 8))
    return out_flat.reshape(N_PAD, OUT_CH)[:N_NODES]


# speedup vs baseline: 30.9016x; 1.0020x over previous
import jax
import jax.numpy as jnp
from jax import lax
from jax.experimental import pallas as pl
from jax.experimental.pallas import tpu as pltpu
from jax.experimental.pallas import tpu_sc as plsc

N_NODES = 10000
N_PAD = 10240
E = 160000
EC = 64
E_ROWS = 2560
E_PAD = E_ROWS * EC
IN_CH = 256
HALF = 128
HIDDEN = 512
OUT_CH = 2
NC = 2
NS = 16
NW = NC * NS
ROWS_A = E_ROWS // NW
ROWS_C = E_ROWS // NS
NPT = N_PAD // NS
RB = 2048
GRID = 5

_mesh = plsc.VectorSubcoreMesh(core_axis_name="c", subcore_axis_name="s")
_sc_params = pltpu.CompilerParams(needs_layout_passes=False)


def _deg_body(dst_hbm, out_hbm, idx_v, hist_v):
    cid = lax.axis_index("c")
    sid = lax.axis_index("s")
    wid = sid * NC + cid
    zeros16 = jnp.zeros((16,), jnp.float32)

    def zb(i, _):
        for k in range(8):
            hist_v[pl.ds(i * 128 + k * 16, 16)] = zeros16
        return 0

    lax.fori_loop(0, N_PAD // 128, zb, 0)
    pltpu.sync_copy(dst_hbm.at[pl.ds(wid * ROWS_A, ROWS_A)], idx_v)
    ones16 = jnp.ones((16,), jnp.float32)

    def body(j, _):
        for k in range(EC // 16):
            idx = idx_v[j, pl.ds(k * 16, 16)]
            plsc.addupdate_scatter(hist_v, [idx], ones16)
        return 0

    lax.fori_loop(0, ROWS_A, body, 0)
    pltpu.sync_copy(hist_v, out_hbm.at[wid])


_deg_call = pl.kernel(
    _deg_body,
    out_type=jax.ShapeDtypeStruct((NW, N_PAD), jnp.float32),
    mesh=_mesh,
    scratch_types=[
        pltpu.VMEM((ROWS_A, EC), jnp.int32),
        pltpu.VMEM((N_PAD,), jnp.float32),
    ],
    compiler_params=_sc_params,
)


def _prep_body(x_ref, degp_ref, dis_ref, y0_ref, y1_ref):
    deg = jnp.sum(degp_ref[...], axis=0) + 1.0
    dis = lax.rsqrt(deg)[:, None]
    dis_ref[...] = dis
    xb = x_ref[...]
    y0_ref[...] = xb[:, :HALF] * dis
    y1_ref[...] = xb[:, HALF:] * dis


_prep_call = pl.pallas_call(
    _prep_body,
    grid=(GRID,),
    in_specs=[
        pl.BlockSpec((RB, IN_CH), lambda i: (i, 0)),
        pl.BlockSpec((NW, RB), lambda i: (0, i)),
    ],
    out_specs=[
        pl.BlockSpec((RB, 1), lambda i: (i, 0)),
        pl.BlockSpec((RB, HALF), lambda i: (i, 0)),
        pl.BlockSpec((RB, HALF), lambda i: (i, 0)),
    ],
    out_shape=[
        jax.ShapeDtypeStruct((N_PAD, 1), jnp.float32),
        jax.ShapeDtypeStruct((N_PAD, HALF), jnp.float32),
        jax.ShapeDtypeStruct((N_PAD, HALF), jnp.float32),
    ],
)


def _agg_body(src_hbm, dst_hbm, y0_hbm, y1_hbm, agg0_hbm, agg1_hbm,
              srcv, dstv, b0, b1, b2, b3, acc_sh,
              g0, g1, g2, g3, s0, s1, s2, s3):
    cid = lax.axis_index("c")
    sid = lax.axis_index("s")
    base = sid * ROWS_C
    hrows = ROWS_C // 4
    bufs = (b0, b1, b2, b3)
    gsems = (g0, g1, g2, g3)
    ssems = (s0, s1, s2, s3)

    zeros16 = jnp.zeros((16,), jnp.float32)

    def zb(i, _):
        for k in range(HALF // 16):
            b0[i, pl.ds(k * 16, 16)] = zeros16
        return 0

    lax.fori_loop(0, EC, zb, 0)
    for k in range(NPT // EC):
        pltpu.async_copy(b0, acc_sh.at[pl.ds(sid * NPT + k * EC, EC)], g0)
    for k in range(NPT // EC):
        pltpu.make_async_copy(b0, acc_sh.at[pl.ds(sid * NPT, EC)], g0).wait()
    plsc.subcore_barrier()

    def do_core(y_hbm):
        for h in range(4):
            pltpu.sync_copy(src_hbm.at[pl.ds(base + h * hrows, hrows)], srcv)
            pltpu.sync_copy(dst_hbm.at[pl.ds(base + h * hrows, hrows)], dstv)
            pltpu.async_copy(y_hbm.at[srcv.at[0]], bufs[0], gsems[0])

            def body(ji, _):
                for b in range(4):
                    j = ji * 4 + b
                    nb_ = (b + 1) % 4

                    @pl.when(j >= 3)
                    def _():
                        pltpu.make_async_copy(
                            bufs[nb_], acc_sh.at[dstv.at[j]], ssems[nb_]
                        ).wait()

                    @pl.when(j + 1 < hrows)
                    def _():
                        pltpu.async_copy(
                            y_hbm.at[srcv.at[j + 1]], bufs[nb_], gsems[nb_])

                    pltpu.make_async_copy(
                        y_hbm.at[srcv.at[j]], bufs[b], gsems[b]).wait()
                    pltpu.async_copy(
                        bufs[b], acc_sh.at[dstv.at[j]], ssems[b], add=True)
                return 0

            lax.fori_loop(0, hrows // 4, body, 0)
            for b in (1, 2, 3):
                pltpu.make_async_copy(
                    bufs[b], acc_sh.at[dstv.at[0]], ssems[b]).wait()

    @pl.when(cid == 0)
    def _():
        do_core(y0_hbm)

    @pl.when(cid == 1)
    def _():
        do_core(y1_hbm)

    plsc.subcore_barrier()
    nb = sid * NPT

    @pl.when(cid == 0)
    def _():
        pltpu.sync_copy(acc_sh.at[pl.ds(nb, NPT)], agg0_hbm.at[pl.ds(nb, NPT)])

    @pl.when(cid == 1)
    def _():
        pltpu.sync_copy(acc_sh.at[pl.ds(nb, NPT)], agg1_hbm.at[pl.ds(nb, NPT)])


_agg_call = pl.kernel(
    _agg_body,
    out_type=(
        jax.ShapeDtypeStruct((N_PAD, HALF), jnp.float32),
        jax.ShapeDtypeStruct((N_PAD, HALF), jnp.float32),
    ),
    mesh=_mesh,
    scratch_types=[
        pltpu.VMEM((ROWS_C // 4, EC), jnp.int32),
        pltpu.VMEM((ROWS_C // 4, EC), jnp.int32),
        pltpu.VMEM((EC, HALF), jnp.float32),
        pltpu.VMEM((EC, HALF), jnp.float32),
        pltpu.VMEM((EC, HALF), jnp.float32),
        pltpu.VMEM((EC, HALF), jnp.float32),
        pltpu.VMEM_SHARED((N_PAD, HALF), jnp.float32),
        pltpu.SemaphoreType.DMA,
        pltpu.SemaphoreType.DMA,
        pltpu.SemaphoreType.DMA,
        pltpu.SemaphoreType.DMA,
        pltpu.SemaphoreType.DMA,
        pltpu.SemaphoreType.DMA,
        pltpu.SemaphoreType.DMA,
        pltpu.SemaphoreType.DMA,
    ],
    compiler_params=_sc_params,
)


def _main_body(dis_ref, agg0_ref, agg1_ref, y0_ref, y1_ref,
               w1_ref, b1_ref, w2_ref, u_ref):
    dis = dis_ref[...]
    z0 = ((agg0_ref[...] + y0_ref[...]) * dis).astype(jnp.bfloat16)
    z1 = ((agg1_ref[...] + y1_ref[...]) * dis).astype(jnp.bfloat16)
    w1 = w1_ref[...].astype(jnp.bfloat16)
    h = jnp.dot(z0, w1[:HALF], preferred_element_type=jnp.float32)
    h = h + jnp.dot(z1, w1[HALF:], preferred_element_type=jnp.float32)
    h = jnp.maximum(h + b1_ref[...], 0.0).astype(jnp.bfloat16)
    w2 = w2_ref[...].astype(jnp.bfloat16)
    u_ref[...] = jnp.dot(h, w2, preferred_element_type=jnp.float32) * dis


_main_call = pl.pallas_call(
    _main_body,
    grid=(GRID,),
    in_specs=[
        pl.BlockSpec((RB, 1), lambda i: (i, 0)),
        pl.BlockSpec((RB, HALF), lambda i: (i, 0)),
        pl.BlockSpec((RB, HALF), lambda i: (i, 0)),
        pl.BlockSpec((RB, HALF), lambda i: (i, 0)),
        pl.BlockSpec((RB, HALF), lambda i: (i, 0)),
        pl.BlockSpec((IN_CH, HIDDEN), lambda i: (0, 0)),
        pl.BlockSpec((1, HIDDEN), lambda i: (0, 0)),
        pl.BlockSpec((HIDDEN, OUT_CH), lambda i: (0, 0)),
    ],
    out_specs=pl.BlockSpec((RB, OUT_CH), lambda i: (i, 0)),
    out_shape=jax.ShapeDtypeStruct((N_PAD, OUT_CH), jnp.float32),
)


def _fin_body(src_hbm, dst_hbm, u_hbm, dis_hbm, b2_hbm, out_hbm,
              srcv, dstv, u_v, p_v, tmp_v, o_v, b2_v, dis_v, sp_sh, sg0):
    cid = lax.axis_index("c")
    sid = lax.axis_index("s")

    @pl.when(cid == 0)
    def _():
        pltpu.async_copy(src_hbm.at[pl.ds(sid * ROWS_C, ROWS_C)], srcv, sg0)
        pltpu.async_copy(dst_hbm.at[pl.ds(sid * ROWS_C, ROWS_C)], dstv, sg0)
        pltpu.async_copy(u_hbm, u_v, sg0)
        nbase = sid * NPT
        pltpu.async_copy(dis_hbm.at[pl.ds(nbase, NPT)], dis_v, sg0)
        pltpu.async_copy(b2_hbm, b2_v, sg0)
        zeros16 = jnp.zeros((16,), jnp.float32)

        def zb(i, _):
            for k in range(8):
                p_v[pl.ds(i * 128 + k * 16, 16)] = zeros16
            return 0

        lax.fori_loop(0, (2 * N_PAD) // 128, zb, 0)
        pltpu.make_async_copy(src_hbm.at[pl.ds(sid * ROWS_C, ROWS_C)], srcv, sg0).wait()
        pltpu.make_async_copy(dst_hbm.at[pl.ds(sid * ROWS_C, ROWS_C)], dstv, sg0).wait()
        pltpu.make_async_copy(u_hbm, u_v, sg0).wait()
        pltpu.make_async_copy(dis_hbm.at[pl.ds(nbase, NPT)], dis_v, sg0).wait()
        pltpu.make_async_copy(b2_hbm, b2_v, sg0).wait()
        ones = jnp.ones((16,), jnp.int32)

        def body(jj, _):
            for r in range(2):
                j = jj * 2 + r
                for k in range(EC // 16):
                    s16 = srcv[j, pl.ds(k * 16, 16)]
                    d16 = dstv[j, pl.ds(k * 16, 16)]
                    s2 = s16 + s16
                    d2 = d16 + d16
                    g0 = plsc.load_gather(u_v, [s2])
                    g1 = plsc.load_gather(u_v, [s2 + ones])
                    plsc.addupdate_scatter(p_v, [d2], g0)
                    plsc.addupdate_scatter(p_v, [d2 + ones], g1)
            return 0

        lax.fori_loop(0, ROWS_C // 2, body, 0)
        pltpu.sync_copy(p_v, sp_sh.at[sid])
        plsc.subcore_barrier()

        fbase = sid * 2 * NPT

        def zb2(i, _):
            for k in range(8):
                o_v[pl.ds(i * 128 + k * 16, 16)] = zeros16
            return 0

        lax.fori_loop(0, (2 * NPT) // 128, zb2, 0)
        for t in range(NS):
            pltpu.sync_copy(sp_sh.at[t, pl.ds(fbase, 2 * NPT)], tmp_v)

            def ab(i, _):
                for k in range(8):
                    o_v[pl.ds(i * 128 + k * 16, 16)] += tmp_v[pl.ds(i * 128 + k * 16, 16)]
                return 0

            lax.fori_loop(0, (2 * NPT) // 128, ab, 0)

        b2v = b2_v[...]
        iota = lax.iota(jnp.int32, 16)
        half_iota = lax.shift_right_logical(iota, 1)

        def fb(i, _):
            pv = o_v[pl.ds(i * 16, 16)]
            uv = u_v[pl.ds(fbase + i * 16, 16)]
            dpair = plsc.load_gather(dis_v, [i * 8 + half_iota])
            o_v[pl.ds(i * 16, 16)] = dpair * (pv + uv) + b2v
            return 0

        lax.fori_loop(0, (2 * NPT) // 16, fb, 0)
        pltpu.sync_copy(o_v, out_hbm.at[pl.ds(fbase, 2 * NPT)])


_fin_call = pl.kernel(
    _fin_body,
    out_type=jax.ShapeDtypeStruct((2 * N_PAD,), jnp.float32),
    mesh=_mesh,
    scratch_types=[
        pltpu.VMEM((ROWS_C, EC), jnp.int32),
        pltpu.VMEM((ROWS_C, EC), jnp.int32),
        pltpu.VMEM((2 * N_PAD,), jnp.float32),
        pltpu.VMEM((2 * N_PAD,), jnp.float32),
        pltpu.VMEM((2 * NPT,), jnp.float32),
        pltpu.VMEM((2 * NPT,), jnp.float32),
        pltpu.VMEM((16,), jnp.float32),
        pltpu.VMEM((NPT,), jnp.float32),
        pltpu.VMEM_SHARED((NS, 2 * N_PAD), jnp.float32),
        pltpu.SemaphoreType.DMA,
    ],
    compiler_params=_sc_params,
)


def kernel(x, edge_index, W1, b1, W2, b2):
    ei = edge_index.astype(jnp.int32)
    src, dst = ei[0], ei[1]
    padi = jnp.arange(E_PAD - E, dtype=jnp.int32)
    pad_src = padi % N_NODES
    pad_dst = N_NODES + padi % (N_PAD - N_NODES)
    srcp = jnp.concatenate([src, pad_src]).reshape(E_ROWS, EC)
    dstp = jnp.concatenate([dst, pad_dst]).reshape(E_ROWS, EC)

    degp = _deg_call(dstp)
    dis, y0, y1 = _prep_call(x, degp)
    agg0, agg1 = _agg_call(srcp, dstp, y0, y1)
    u = _main_call(dis, agg0, agg1, y0, y1, W1, b1.reshape(1, HIDDEN), W2)
    out_flat = _fin_call(srcp, dstp, u.reshape(2 * N_PAD),
                         dis.reshape(N_PAD), jnp.tile(b2, 8))
    return out_flat.reshape(N_PAD, OUT_CH)[:N_NODES]
